# baseline (speedup 1.0000x reference)
import functools

import jax
import jax.numpy as jnp
from jax import lax
from jax.experimental import pallas as pl
from jax.experimental.pallas import tpu as pltpu
from jax.experimental.pallas import tpu_sc as plsc

NUM_VOCAB = 100000
EMBED_DIM = 128

_INFO = plsc.get_sparse_core_info()
_NC, _NS = _INFO.num_cores, _INFO.num_subcores
_NW = _NC * _NS

_CHUNK = 128
_D = 5
_K = 2


@functools.partial(jax.jit, static_argnums=(2,))
def _gather_rows(idx_flat, table, n_rows):
    b_per_w = n_rows // _NW
    n_chunks = b_per_w // _CHUNK
    n_super = n_chunks // _D
    mesh = plsc.VectorSubcoreMesh(core_axis_name="c", subcore_axis_name="s")

    @functools.partial(
        pl.kernel,
        mesh=mesh,
        out_type=jax.ShapeDtypeStruct((n_rows, EMBED_DIM), jnp.float32),
        scratch_types=[
            pltpu.VMEM((b_per_w,), jnp.int32),
            pltpu.VMEM((_D, _CHUNK, EMBED_DIM), jnp.float32),
            pltpu.SemaphoreType.DMA((_D,)),
            pltpu.SemaphoreType.DMA((_D,)),
        ],
        compiler_params=pltpu.CompilerParams(use_tc_tiling_on_sc=True),
    )
    def k(idx_hbm, table_hbm, out_hbm, idx_v, rows_v, gsem, wsem):
        wid = lax.axis_index("s") * _NC + lax.axis_index("c")
        w_base = wid * b_per_w
        pltpu.sync_copy(idx_hbm.at[pl.ds(w_base, b_per_w)], idx_v)

        def gather(chunk, b):
            return pltpu.make_async_copy(
                table_hbm.at[idx_v.at[pl.ds(chunk * _CHUNK, _CHUNK)]],
                rows_v.at[b],
                gsem.at[b],
            )

        def writeback(chunk, b):
            return pltpu.make_async_copy(
                rows_v.at[b],
                out_hbm.at[pl.ds(w_base + chunk * _CHUNK, _CHUNK)],
                wsem.at[b],
            )

        for b in range(_K):
            gather(b, b).start()

        def super_step(s, carry):
            for b in range(_D):
                i = s * _D + b
                gather(i, b).wait()
                writeback(i, b).start()
                j = i + _K
                jb = (b + _K) % _D

                @pl.when(j < n_chunks)
                def _():
                    @pl.when(j >= _D)
                    def _():
                        writeback(j - _D, jb).wait()

                    gather(j, jb).start()

            return carry

        lax.fori_loop(0, n_super, super_step, 0)

        for b in range(_D):
            writeback(n_chunks - _D + b, b).wait()

    return k(idx_flat, table)


def kernel(input, table):
    b, h = input.shape
    idx_flat = input.T.reshape(h * b).astype(jnp.int32)
    out = _gather_rows(idx_flat, table, h * b)
    return out.reshape(h, b, EMBED_DIM).transpose(1, 0, 2)


# speedup vs baseline: 1.0029x; 1.0029x over previous
import functools

import jax
import jax.numpy as jnp
from jax import lax
from jax.experimental import pallas as pl
from jax.experimental.pallas import tpu as pltpu
from jax.experimental.pallas import tpu_sc as plsc

NUM_VOCAB = 100000
EMBED_DIM = 128

_INFO = plsc.get_sparse_core_info()
_NC, _NS = _INFO.num_cores, _INFO.num_subcores
_NW = _NC * _NS

_CHUNK = 64
_D = 10
_K = 5


@functools.partial(jax.jit, static_argnums=(2,))
def _gather_rows(idx_flat, table, n_rows):
    b_per_w = n_rows // _NW
    n_chunks = b_per_w // _CHUNK
    n_super = n_chunks // _D
    mesh = plsc.VectorSubcoreMesh(core_axis_name="c", subcore_axis_name="s")

    @functools.partial(
        pl.kernel,
        mesh=mesh,
        out_type=jax.ShapeDtypeStruct((n_rows, EMBED_DIM), jnp.float32),
        scratch_types=[
            pltpu.VMEM((b_per_w,), jnp.int32),
            pltpu.VMEM((_D, _CHUNK, EMBED_DIM), jnp.float32),
            pltpu.SemaphoreType.DMA((_D,)),
            pltpu.SemaphoreType.DMA((_D,)),
        ],
        compiler_params=pltpu.CompilerParams(use_tc_tiling_on_sc=True),
    )
    def k(idx_hbm, table_hbm, out_hbm, idx_v, rows_v, gsem, wsem):
        wid = lax.axis_index("s") * _NC + lax.axis_index("c")
        w_base = wid * b_per_w
        pltpu.sync_copy(idx_hbm.at[pl.ds(w_base, b_per_w)], idx_v)

        def gather(chunk, b):
            return pltpu.make_async_copy(
                table_hbm.at[idx_v.at[pl.ds(chunk * _CHUNK, _CHUNK)]],
                rows_v.at[b],
                gsem.at[b],
            )

        def writeback(chunk, b):
            return pltpu.make_async_copy(
                rows_v.at[b],
                out_hbm.at[pl.ds(w_base + chunk * _CHUNK, _CHUNK)],
                wsem.at[b],
            )

        for b in range(_K):
            gather(b, b).start()

        def super_step(s, carry):
            for b in range(_D):
                i = s * _D + b
                gather(i, b).wait()
                writeback(i, b).start()
                j = i + _K
                jb = (b + _K) % _D

                @pl.when(j < n_chunks)
                def _():
                    @pl.when(j >= _D)
                    def _():
                        writeback(j - _D, jb).wait()

                    gather(j, jb).start()

            return carry

        lax.fori_loop(0, n_super, super_step, 0)

        for b in range(_D):
            writeback(n_chunks - _D + b, b).wait()

    return k(idx_flat, table)


def kernel(input, table):
    b, h = input.shape
    idx_flat = input.T.reshape(h * b).astype(jnp.int32)
    out = _gather_rows(idx_flat, table, h * b)
    return out.reshape(h, b, EMBED_DIM).transpose(1, 0, 2)


# speedup vs baseline: 1.0131x; 1.0101x over previous
import functools

import jax
import jax.numpy as jnp
from jax import lax
from jax.experimental import pallas as pl
from jax.experimental.pallas import tpu as pltpu
from jax.experimental.pallas import tpu_sc as plsc

NUM_VOCAB = 100000
EMBED_DIM = 128

_INFO = plsc.get_sparse_core_info()
_NC, _NS = _INFO.num_cores, _INFO.num_subcores
_NW = _NC * _NS

_CHUNK = 80
_D = 10
_K = 7


@functools.partial(jax.jit, static_argnums=(2,))
def _gather_rows(idx_flat, table, n_rows):
    b_per_w = n_rows // _NW
    n_chunks = b_per_w // _CHUNK
    n_super = n_chunks // _D
    mesh = plsc.VectorSubcoreMesh(core_axis_name="c", subcore_axis_name="s")

    @functools.partial(
        pl.kernel,
        mesh=mesh,
        out_type=jax.ShapeDtypeStruct((n_rows, EMBED_DIM), jnp.float32),
        scratch_types=[
            pltpu.VMEM((b_per_w,), jnp.int32),
            pltpu.VMEM((_D, _CHUNK, EMBED_DIM), jnp.float32),
            pltpu.SemaphoreType.DMA((_D,)),
            pltpu.SemaphoreType.DMA((_D,)),
        ],
        compiler_params=pltpu.CompilerParams(use_tc_tiling_on_sc=True),
    )
    def k(idx_hbm, table_hbm, out_hbm, idx_v, rows_v, gsem, wsem):
        wid = lax.axis_index("s") * _NC + lax.axis_index("c")
        w_base = wid * b_per_w
        pltpu.sync_copy(idx_hbm.at[pl.ds(w_base, b_per_w)], idx_v)

        def gather(chunk, b):
            return pltpu.make_async_copy(
                table_hbm.at[idx_v.at[pl.ds(chunk * _CHUNK, _CHUNK)]],
                rows_v.at[b],
                gsem.at[b],
            )

        def writeback(chunk, b):
            return pltpu.make_async_copy(
                rows_v.at[b],
                out_hbm.at[pl.ds(w_base + chunk * _CHUNK, _CHUNK)],
                wsem.at[b],
            )

        for b in range(_K):
            gather(b, b).start()

        def super_step(s, carry):
            for b in range(_D):
                i = s * _D + b
                gather(i, b).wait()
                writeback(i, b).start()
                j = i + _K
                jb = (b + _K) % _D

                @pl.when(j < n_chunks)
                def _():
                    @pl.when(j >= _D)
                    def _():
                        writeback(j - _D, jb).wait()

                    gather(j, jb).start()

            return carry

        lax.fori_loop(0, n_super, super_step, 0)

        for b in range(_D):
            writeback(n_chunks - _D + b, b).wait()

    return k(idx_flat, table)


def kernel(input, table):
    b, h = input.shape
    idx_flat = input.T.reshape(h * b).astype(jnp.int32)
    out = _gather_rows(idx_flat, table, h * b)
    return out.reshape(h, b, EMBED_DIM).transpose(1, 0, 2)
